# baseline (speedup 1.0000x reference)
import functools

import jax
import jax.numpy as jnp
from jax import lax
from jax.experimental import pallas as pl
from jax.experimental.pallas import tpu as pltpu
from jax.experimental.pallas import tpu_sc as plsc

_BLK = 128
_NBUF = 2
_TS = _BLK + 1


@functools.cache
def _make_gather(batch, seq, V, D, n_workers, nc):
    n_blk = batch // _BLK
    n_tasks = seq * n_blk
    tpw = n_tasks // n_workers
    assert tpw * n_workers == n_tasks and tpw % _NBUF == 0
    d_oct = D // 8
    mesh = plsc.VectorSubcoreMesh(core_axis_name="c", subcore_axis_name="s")

    scratch = (
        [pltpu.VMEM((_BLK,), jnp.int32) for _ in range(_NBUF)]
        + [pltpu.VMEM((_BLK, D), jnp.float32) for _ in range(_NBUF)]
        + [pltpu.VMEM((D, _TS), jnp.float32) for _ in range(_NBUF)]
        + [pltpu.SemaphoreType.DMA] * (3 * _NBUF)
    )

    @functools.partial(
        pl.kernel,
        mesh=mesh,
        out_type=jax.ShapeDtypeStruct((seq, d_oct, n_blk, 8, _BLK), jnp.float32),
        scratch_types=scratch,
        compiler_params=pltpu.CompilerParams(
            use_tc_tiling_on_sc=False, needs_layout_passes=False
        ),
    )
    def gather_kernel(idx_hbm, table_hbm, out_hbm, *bufs):
        idx_v = bufs[0:_NBUF]
        rows_v = bufs[_NBUF : 2 * _NBUF]
        tile_v = bufs[2 * _NBUF : 3 * _NBUF]
        s_i = bufs[3 * _NBUF : 4 * _NBUF]
        s_g = bufs[4 * _NBUF : 5 * _NBUF]
        s_o = bufs[5 * _NBUF : 6 * _NBUF]

        wid = lax.axis_index("s") * nc + lax.axis_index("c")
        g0 = wid * tpw

        iota = lax.iota(jnp.int32, 16)
        d_ids = [iota + 16 * k for k in range(D // 16)]

        def start_idx(t, b):
            return pltpu.make_async_copy(
                idx_hbm.at[pl.ds((g0 + t) * _BLK, _BLK)], idx_v[b], s_i[b]
            )

        def gather(b):
            return pltpu.make_async_copy(table_hbm.at[idx_v[b]], rows_v[b], s_g[b])

        def transpose(b):
            @plsc.parallel_loop(0, _BLK, unroll=16)
            def tbody(r):
                r_ids = jnp.full((16,), 0, jnp.int32) + r
                for k in range(D // 16):
                    vals = rows_v[b][r, pl.ds(16 * k, 16)]
                    plsc.store_scatter(tile_v[b], [d_ids[k], r_ids], vals)

        def write_out(t, b, wait):
            g = g0 + t
            s = g // n_blk
            blk = g - s * n_blk
            for o in range(d_oct):
                cp = pltpu.make_async_copy(
                    tile_v[b].at[pl.ds(8 * o, 8), pl.ds(0, _BLK)],
                    out_hbm.at[s, o, blk],
                    s_o[b],
                )
                cp.wait() if wait else cp.start()

        for b in range(_NBUF):
            start_idx(b, b).start()
        for b in range(_NBUF):
            start_idx(b, b).wait()
            gather(b).start()

        def body(p, carry):
            for b in range(_NBUF):
                t = p * _NBUF + b
                gather(b).wait()
                start_idx(t + _NBUF, b).start()

                @pl.when(p > 0)
                def _():
                    write_out(t - _NBUF, b, wait=True)

                transpose(b)
                write_out(t, b, wait=False)
                start_idx(t + _NBUF, b).wait()
                gather(b).start()
            return carry

        lax.fori_loop(0, tpw // _NBUF - 1, body, 0)

        for b in range(_NBUF):
            t = tpw - _NBUF + b
            gather(b).wait()
            write_out(t - _NBUF, b, wait=True)
            transpose(b)
            write_out(t, b, wait=False)
        for b in range(_NBUF):
            write_out(tpw - _NBUF + b, b, wait=True)

    return gather_kernel


def kernel(indices, table):
    batch, seq = indices.shape
    vocab, dim = table.shape
    info = plsc.get_sparse_core_info()
    n_workers = info.num_cores * info.num_subcores
    idx_sm = jnp.transpose(indices).reshape(-1)
    out5 = _make_gather(batch, seq, vocab, dim, n_workers, info.num_cores)(
        idx_sm, table
    )
    return jnp.transpose(out5, (2, 4, 0, 1, 3)).reshape(batch, seq, dim)


# speedup vs baseline: 1.1638x; 1.1638x over previous
import functools

import jax
import jax.numpy as jnp
from jax import lax
from jax.experimental import pallas as pl
from jax.experimental.pallas import tpu as pltpu
from jax.experimental.pallas import tpu_sc as plsc

_BLK = 128
_NBUF = 4
_TS = _BLK + 1


@functools.cache
def _make_gather(batch, seq, V, D, n_workers, nc):
    n_blk = batch // _BLK
    n_tasks = seq * n_blk
    tpw = n_tasks // n_workers
    assert tpw * n_workers == n_tasks and tpw % _NBUF == 0
    d_oct = D // 8
    mesh = plsc.VectorSubcoreMesh(core_axis_name="c", subcore_axis_name="s")

    scratch = (
        [pltpu.VMEM((_BLK,), jnp.int32) for _ in range(_NBUF)]
        + [pltpu.VMEM((_BLK, D), jnp.float32) for _ in range(_NBUF)]
        + [pltpu.VMEM((D, _TS), jnp.float32) for _ in range(_NBUF)]
        + [pltpu.SemaphoreType.DMA] * (3 * _NBUF)
    )

    @functools.partial(
        pl.kernel,
        mesh=mesh,
        out_type=jax.ShapeDtypeStruct((seq, d_oct, n_blk, 8, _BLK), jnp.float32),
        scratch_types=scratch,
        compiler_params=pltpu.CompilerParams(
            use_tc_tiling_on_sc=False, needs_layout_passes=False
        ),
    )
    def gather_kernel(idx_hbm, table_hbm, out_hbm, *bufs):
        idx_v = bufs[0:_NBUF]
        rows_v = bufs[_NBUF : 2 * _NBUF]
        tile_v = bufs[2 * _NBUF : 3 * _NBUF]
        s_i = bufs[3 * _NBUF : 4 * _NBUF]
        s_g = bufs[4 * _NBUF : 5 * _NBUF]
        s_o = bufs[5 * _NBUF : 6 * _NBUF]

        wid = lax.axis_index("s") * nc + lax.axis_index("c")
        g0 = wid * tpw

        iota = lax.iota(jnp.int32, 16)
        d_ids = [iota + 16 * k for k in range(D // 16)]

        def start_idx(t, b):
            return pltpu.make_async_copy(
                idx_hbm.at[pl.ds((g0 + t) * _BLK, _BLK)], idx_v[b], s_i[b]
            )

        def gather(b):
            return pltpu.make_async_copy(table_hbm.at[idx_v[b]], rows_v[b], s_g[b])

        def transpose(b):
            @plsc.parallel_loop(0, _BLK, unroll=8)
            def tbody(r):
                r_ids = jnp.full((16,), 0, jnp.int32) + r
                for k in range(D // 16):
                    vals = rows_v[b][r, pl.ds(16 * k, 16)]
                    plsc.store_scatter(tile_v[b], [d_ids[k], r_ids], vals)

        def write_out(t, b, wait):
            g = g0 + t
            s = g // n_blk
            blk = g - s * n_blk
            for o in range(d_oct):
                cp = pltpu.make_async_copy(
                    tile_v[b].at[pl.ds(8 * o, 8), pl.ds(0, _BLK)],
                    out_hbm.at[s, o, blk],
                    s_o[b],
                )
                cp.wait() if wait else cp.start()

        for b in range(_NBUF):
            start_idx(b, b).start()
        for b in range(_NBUF):
            start_idx(b, b).wait()
            gather(b).start()

        def body(p, carry):
            for b in range(_NBUF):
                t = p * _NBUF + b
                gather(b).wait()
                start_idx(t + _NBUF, b).start()

                @pl.when(p > 0)
                def _():
                    write_out(t - _NBUF, b, wait=True)

                transpose(b)
                write_out(t, b, wait=False)
                start_idx(t + _NBUF, b).wait()
                gather(b).start()
            return carry

        lax.fori_loop(0, tpw // _NBUF - 1, body, 0)

        for b in range(_NBUF):
            t = tpw - _NBUF + b
            gather(b).wait()
            write_out(t - _NBUF, b, wait=True)
            transpose(b)
            write_out(t, b, wait=False)
        for b in range(_NBUF):
            write_out(tpw - _NBUF + b, b, wait=True)

    return gather_kernel


def kernel(indices, table):
    batch, seq = indices.shape
    vocab, dim = table.shape
    info = plsc.get_sparse_core_info()
    n_workers = info.num_cores * info.num_subcores
    idx_sm = jnp.transpose(indices).reshape(-1)
    out5 = _make_gather(batch, seq, vocab, dim, n_workers, info.num_cores)(
        idx_sm, table
    )
    return jnp.transpose(out5, (2, 4, 0, 1, 3)).reshape(batch, seq, dim)
